# Initial kernel scaffold; baseline (speedup 1.0000x reference)
import jax
import jax.numpy as jnp
from jax.experimental import pallas as pl


def kernel(relpos, query_features, scaling, query_batch_cnt, table_x, table_y, table_z):
    raise NotImplementedError("write your pallas kernel here")


# speedup vs baseline: 6.5803x; 6.5803x over previous
import functools

import jax
import jax.numpy as jnp
import numpy as np
from jax import lax
from jax.experimental import pallas as pl
from jax.experimental.pallas import tpu as pltpu
from jax.experimental.pallas import tpu_sc as plsc

NHEAD = 8
HDIM = 12
DIR_NUM = 3
PER_DIR = 4
LOCAL = 32
N_QUERY = 65536
QUAN = np.float32(0.4)
_PC_RANGE = np.array([-75.2, -75.2, -2.0, 75.2, 75.2, 4.0], dtype=np.float64).reshape(2, 3)
MAXD = (_PC_RANGE[1] - _PC_RANGE[0]).astype(np.float32)
TSIZES = tuple(int(x) for x in (np.ceil(MAXD.astype(np.float64) * 2 / 0.4).astype(np.int64) + 1))

ROW = NHEAD * PER_DIR
TOFF = (0, TSIZES[0] * ROW, (TSIZES[0] + TSIZES[1]) * ROW)
TAB_WORDS = (TSIZES[0] + TSIZES[1] + TSIZES[2]) * ROW

NC, NS, LANES = 2, 16, 16
NW = NC * NS
QPW = N_QUERY // NW
CHUNK = 64
RP_W = LOCAL * DIR_NUM
QF_W = NHEAD * HDIM
OUT_W = LOCAL * NHEAD


def _sc_body(rp_hbm, qf_hbm, tab_hbm, out_hbm, tab_v, rp_v, qf_v, out_v):
    wid = lax.axis_index("s") * NC + lax.axis_index("c")
    pltpu.sync_copy(tab_hbm, tab_v)

    iota = lax.iota(jnp.int32, LANES)
    lane3 = iota * 3
    lane8 = iota * 8

    def q_body(qi, _):
        roff = qi * RP_W
        bases = []
        for d in range(DIR_NUM):
            for lg in range(2):
                rpx = plsc.load_gather(rp_v, [lane3 + (roff + lg * 48 + d)])
                t = (rpx + MAXD[d]) / QUAN
                idx = jnp.clip(t.astype(jnp.int32), 0, TSIZES[d] - 1)
                bases.append(idx * ROW + TOFF[d])
        acc = [jnp.zeros((LANES,), jnp.float32) for _ in range(NHEAD * 2)]
        qoff = qi * QF_W
        for d in range(DIR_NUM):
            for h in range(NHEAD):
                for c in range(PER_DIR):
                    lin = qoff + h * HDIM + d * PER_DIR + c
                    sp = plsc.load_gather(qf_v, [jnp.full((LANES,), 0, jnp.int32) + lin])
                    w = h * PER_DIR + c
                    for lg in range(2):
                        g = plsc.load_gather(tab_v, [bases[d * 2 + lg] + w])
                        acc[h * 2 + lg] = acc[h * 2 + lg] + g * sp
        obase = qi * OUT_W
        for h in range(NHEAD):
            for lg in range(2):
                plsc.store_scatter(out_v, [lane8 + (obase + lg * 128 + h)], acc[h * 2 + lg])
        return 0

    def chunk_body(ci, _):
        qbase = wid * QPW + ci * CHUNK
        pltpu.sync_copy(rp_hbm.at[pl.ds(qbase * RP_W, CHUNK * RP_W)], rp_v)
        pltpu.sync_copy(qf_hbm.at[pl.ds(qbase * QF_W, CHUNK * QF_W)], qf_v)
        lax.fori_loop(0, CHUNK, q_body, 0)
        pltpu.sync_copy(out_v, out_hbm.at[pl.ds(qbase * OUT_W, CHUNK * OUT_W)])
        return 0

    lax.fori_loop(0, QPW // CHUNK, chunk_body, 0)


@jax.jit
def _run(rp_flat, qf_flat, tab_flat):
    mesh = plsc.VectorSubcoreMesh(core_axis_name="c", subcore_axis_name="s")
    f = functools.partial(
        pl.kernel,
        out_type=jax.ShapeDtypeStruct((N_QUERY * OUT_W,), jnp.float32),
        mesh=mesh,
        scratch_types=[
            pltpu.VMEM((TAB_WORDS,), jnp.float32),
            pltpu.VMEM((CHUNK * RP_W,), jnp.float32),
            pltpu.VMEM((CHUNK * QF_W,), jnp.float32),
            pltpu.VMEM((CHUNK * OUT_W,), jnp.float32),
        ],
        compiler_params=pltpu.CompilerParams(needs_layout_passes=False),
    )(_sc_body)
    return f(rp_flat, qf_flat, tab_flat)


def kernel(relpos, query_features, scaling, query_batch_cnt, table_x, table_y, table_z):
    del query_batch_cnt
    tab_flat = jnp.concatenate(
        [table_x.reshape(-1), table_y.reshape(-1), table_z.reshape(-1)]
    ) * scaling
    out = _run(relpos.reshape(-1), query_features.reshape(-1), tab_flat)
    return out.reshape(N_QUERY, LOCAL, NHEAD)
